# baseline (speedup 1.0000x reference)
import functools

import jax
import jax.numpy as jnp
from jax import lax
from jax.experimental import pallas as pl
from jax.experimental.pallas import tpu as pltpu
from jax.experimental.pallas import tpu_sc as plsc

D = 768
H = 768
N = 8
NG = 7
M = 2048
TM = 256
T = M // TM
NITEMS = T + NG - 1

NW = 16
CHUNK = M // NW
LANES = 16


def _router_kernel(x_ref, wg_ref, ids_ref):
    x = x_ref[0]
    logits = jnp.dot(x, wg_ref[...], preferred_element_type=jnp.float32)
    cols = jax.lax.broadcasted_iota(jnp.int32, logits.shape, 1)
    ml = jnp.where(cols < NG, logits, -jnp.inf)
    mx = jnp.max(ml, axis=1, keepdims=True)
    ids_ref[0, 0, :] = jnp.min(jnp.where(ml >= mx, cols, NG), axis=1).astype(jnp.int32)


def _shared_kernel(x_ref, w1_ref, w2_ref, ysh_ref):
    x = x_ref[...]
    z = jnp.dot(x, w1_ref[0], preferred_element_type=jnp.float32)
    a = z[:, :H]
    b = z[:, H:]
    h = a * b * jax.nn.sigmoid(b)
    ysh_ref[...] = jnp.dot(h, w2_ref[0], preferred_element_type=jnp.float32)


def _gmm_kernel(meta_ref, xs_ref, w1_ref, w2_ref, out_ref):
    w = pl.program_id(0)
    lo = meta_ref[2 * LANES + w]
    hi = meta_ref[3 * LANES + w]
    init = meta_ref[4 * LANES + w]

    @pl.when(init == 1)
    def _():
        out_ref[...] = jnp.zeros_like(out_ref)

    @pl.when(lo < hi)
    def _():
        x = xs_ref[...]
        z = jnp.dot(x, w1_ref[0], preferred_element_type=jnp.float32)
        a = z[:, :H]
        b = z[:, H:]
        h = a * b * jax.nn.sigmoid(b)
        o = jnp.dot(h, w2_ref[0], preferred_element_type=jnp.float32)
        rows = jax.lax.broadcasted_iota(jnp.int32, (TM, 1), 0)
        keep = (rows >= lo) & (rows < hi)
        out_ref[...] += jnp.where(keep, o, 0.0)


def _sc_dispatch_body(ids_hbm, x_hbm, xs_hbm, pos_hbm, meta_hbm,
                      ids_v, x_v, pos_v, hist_v, hist_all_v,
                      mt_tile_v, mt_exp_v, mt_lo_v, mt_hi_v, mt_init_v,
                      hist_sh, sem_x, sem_s):
    wid = lax.axis_index("s")
    base = wid * CHUNK
    lane = lax.iota(jnp.int32, LANES)
    zeros = jnp.zeros((LANES,), jnp.int32)

    cp_x = pltpu.async_copy(x_hbm.at[pl.ds(base, CHUNK), :], x_v, sem_x)
    pltpu.sync_copy(ids_hbm.at[pl.ds(base, CHUNK)], ids_v)

    hist = zeros
    for k in range(CHUNK // LANES):
        idv = ids_v[pl.ds(k * LANES, LANES)]
        for e in range(NG):
            cnt = jnp.sum(jnp.where(idv == e, 1, 0))
            hist = hist + jnp.where(lane == e, cnt, 0)
    hist_v[...] = hist

    pltpu.sync_copy(hist_v, hist_sh.at[pl.ds(wid * LANES, LANES)])
    plsc.subcore_barrier()
    pltpu.sync_copy(hist_sh, hist_all_v)

    tot = zeros
    pre = zeros
    for w2 in range(NW):
        row = hist_all_v[pl.ds(w2 * LANES, LANES)]
        tot = tot + row
        sel = jnp.full((LANES,), w2, jnp.int32) < wid
        pre = pre + jnp.where(sel, row, zeros)
    c_excl = plsc.cumsum(tot) - tot
    carry = c_excl + pre

    @pl.when(wid == 0)
    def _():
        mt_tile_v[...] = jnp.full((LANES,), T - 1, jnp.int32)
        mt_exp_v[...] = jnp.full((LANES,), NG - 1, jnp.int32)
        mt_lo_v[...] = zeros
        mt_hi_v[...] = zeros
        mt_init_v[...] = zeros
        cs = [jnp.sum(jnp.where(lane == e, c_excl, zeros)) for e in range(NG)]
        cs.append(jnp.asarray(M, jnp.int32))
        carry_s = jnp.asarray(0, jnp.int32)
        for v in range((T * NG + LANES - 1) // LANES):
            p = lane + v * LANES
            tt_v = p // NG
            ee_v = p % NG
            ce_v = zeros
            ce1_v = zeros
            for e in range(NG):
                ce_v = ce_v + jnp.where(ee_v == e, cs[e], 0)
                ce1_v = ce1_v + jnp.where(ee_v == e, cs[e + 1], 0)
            touched = (ce_v < (tt_v + 1) * TM) & (ce1_v > tt_v * TM)
            ti = jnp.where(touched, 1, 0)
            s_p = plsc.cumsum(ti) - ti + carry_s
            lo_v = jnp.maximum(ce_v - tt_v * TM, 0)
            hi_v = jnp.minimum(ce1_v - tt_v * TM, TM)
            plsc.store_scatter(mt_tile_v, [s_p], tt_v, mask=touched)
            plsc.store_scatter(mt_exp_v, [s_p], ee_v, mask=touched)
            plsc.store_scatter(mt_lo_v, [s_p], lo_v, mask=touched)
            plsc.store_scatter(mt_hi_v, [s_p], hi_v, mask=touched)
            plsc.store_scatter(mt_init_v, [s_p],
                               jnp.where(lo_v == 0, 1, 0), mask=touched)
            carry_s = carry_s + jnp.sum(ti)
        pltpu.sync_copy(mt_tile_v, meta_hbm.at[pl.ds(0 * LANES, LANES)])
        pltpu.sync_copy(mt_exp_v, meta_hbm.at[pl.ds(1 * LANES, LANES)])
        pltpu.sync_copy(mt_lo_v, meta_hbm.at[pl.ds(2 * LANES, LANES)])
        pltpu.sync_copy(mt_hi_v, meta_hbm.at[pl.ds(3 * LANES, LANES)])
        pltpu.sync_copy(mt_init_v, meta_hbm.at[pl.ds(4 * LANES, LANES)])

    for k in range(CHUNK // LANES):
        idv = ids_v[pl.ds(k * LANES, LANES)]
        posk = zeros
        vcnt = zeros
        for e in range(NG):
            mi = jnp.where(idv == e, 1, 0)
            pref = plsc.cumsum(mi) - mi
            ce = jnp.sum(jnp.where(lane == e, carry, zeros))
            posk = jnp.where(idv == e, ce + pref, posk)
            vcnt = vcnt + jnp.where(lane == e, jnp.sum(mi), 0)
        pos_v[pl.ds(k * LANES, LANES)] = posk
        carry = carry + vcnt

    cp_x.wait()
    pltpu.async_copy(x_v, xs_hbm.at[pos_v], sem_s).wait()
    pltpu.sync_copy(pos_v, pos_hbm.at[pl.ds(base, CHUNK)])


_sc_dispatch = functools.partial(
    pl.kernel,
    out_type=[
        jax.ShapeDtypeStruct((M, D), jnp.float32),
        jax.ShapeDtypeStruct((M,), jnp.int32),
        jax.ShapeDtypeStruct((5 * LANES,), jnp.int32),
    ],
    mesh=plsc.VectorSubcoreMesh(
        core_axis_name="c", subcore_axis_name="s", num_cores=1),
    compiler_params=pltpu.CompilerParams(needs_layout_passes=False),
    scratch_types=[
        pltpu.VMEM((CHUNK,), jnp.int32),
        pltpu.VMEM((CHUNK, D), jnp.float32),
        pltpu.VMEM((CHUNK,), jnp.int32),
        pltpu.VMEM((LANES,), jnp.int32),
        pltpu.VMEM((NW * LANES,), jnp.int32),
        pltpu.VMEM((LANES,), jnp.int32),
        pltpu.VMEM((LANES,), jnp.int32),
        pltpu.VMEM((LANES,), jnp.int32),
        pltpu.VMEM((LANES,), jnp.int32),
        pltpu.VMEM((LANES,), jnp.int32),
        pltpu.VMEM_SHARED((NW * LANES,), jnp.int32),
        pltpu.SemaphoreType.DMA,
        pltpu.SemaphoreType.DMA,
    ],
)(_sc_dispatch_body)


@jax.jit
def _run(x_BSD, Wg_DN, Wl1_ND2H, Wl2_NHD):
    ids3 = pl.pallas_call(
        _router_kernel,
        grid=(1,),
        in_specs=[
            pl.BlockSpec((1, M, D), lambda t: (0, 0, 0)),
            pl.BlockSpec((D, N), lambda t: (0, 0)),
        ],
        out_specs=pl.BlockSpec((1, 1, M), lambda t: (0, 0, 0)),
        out_shape=jax.ShapeDtypeStruct((1, 1, M), jnp.int32),
    )(x_BSD, Wg_DN)
    ids = ids3.reshape(M)

    x2 = x_BSD.reshape(M, D)
    xs, pos, meta = _sc_dispatch(ids, x2)

    y_sh = pl.pallas_call(
        _shared_kernel,
        grid=(T,),
        in_specs=[
            pl.BlockSpec((TM, D), lambda t: (t, 0)),
            pl.BlockSpec((1, D, 2 * H), lambda t: (NG, 0, 0)),
            pl.BlockSpec((1, H, D), lambda t: (NG, 0, 0)),
        ],
        out_specs=pl.BlockSpec((TM, D), lambda t: (t, 0)),
        out_shape=jax.ShapeDtypeStruct((M, D), jnp.float32),
    )(x2, Wl1_ND2H, Wl2_NHD)

    y_rs = pl.pallas_call(
        _gmm_kernel,
        grid_spec=pltpu.PrefetchScalarGridSpec(
            num_scalar_prefetch=1,
            grid=(LANES,),
            in_specs=[
                pl.BlockSpec((TM, D), lambda w, m: (m[w], 0)),
                pl.BlockSpec((1, D, 2 * H), lambda w, m: (m[LANES + w], 0, 0)),
                pl.BlockSpec((1, H, D), lambda w, m: (m[LANES + w], 0, 0)),
            ],
            out_specs=pl.BlockSpec((TM, D), lambda w, m: (m[w], 0)),
        ),
        out_shape=jax.ShapeDtypeStruct((M, D), jnp.float32),
    )(meta, xs, Wl1_ND2H, Wl2_NHD)

    y = y_sh + y_rs.at[pos].get(mode="promise_in_bounds")
    return y.reshape(x_BSD.shape)


def kernel(x_BSD, Wg_DN, Wl1_ND2H, Wl2_NHD):
    return _run(x_BSD, Wg_DN, Wl1_ND2H, Wl2_NHD)


# speedup vs baseline: 1.0244x; 1.0244x over previous
import functools

import jax
import jax.numpy as jnp
from jax import lax
from jax.experimental import pallas as pl
from jax.experimental.pallas import tpu as pltpu
from jax.experimental.pallas import tpu_sc as plsc

D = 768
H = 768
N = 8
NG = 7
M = 2048
TM = 256
T = M // TM
NITEMS = T + NG - 1

NW = 16
CHUNK = M // NW
LANES = 16


def _router_kernel(x_ref, wg_ref, ids_ref):
    x = x_ref[0]
    logits = jnp.dot(x, wg_ref[...], preferred_element_type=jnp.float32)
    cols = jax.lax.broadcasted_iota(jnp.int32, logits.shape, 1)
    ml = jnp.where(cols < NG, logits, -jnp.inf)
    mx = jnp.max(ml, axis=1, keepdims=True)
    ids_ref[0, 0, :] = jnp.min(jnp.where(ml >= mx, cols, NG), axis=1).astype(jnp.int32)


def _shared_kernel(x_ref, w1_ref, w2_ref, ysh_ref):
    x = x_ref[...]
    z = jnp.dot(x, w1_ref[0], preferred_element_type=jnp.float32)
    a = z[:, :H]
    b = z[:, H:]
    h = a * b * jax.nn.sigmoid(b)
    ysh_ref[...] = jnp.dot(h, w2_ref[0], preferred_element_type=jnp.float32)


def _gmm_kernel(meta_ref, xs_ref, yss_ref, w1_ref, w2_ref, out_ref):
    w = pl.program_id(0)
    lo = meta_ref[2 * LANES + w]
    hi = meta_ref[3 * LANES + w]
    init = meta_ref[4 * LANES + w]

    @pl.when(init == 1)
    def _():
        out_ref[...] = yss_ref[...]

    @pl.when(lo < hi)
    def _():
        x = xs_ref[...]
        z = jnp.dot(x, w1_ref[0], preferred_element_type=jnp.float32)
        a = z[:, :H]
        b = z[:, H:]
        h = a * b * jax.nn.sigmoid(b)
        o = jnp.dot(h, w2_ref[0], preferred_element_type=jnp.float32)
        rows = jax.lax.broadcasted_iota(jnp.int32, (TM, 1), 0)
        keep = (rows >= lo) & (rows < hi)
        out_ref[...] += jnp.where(keep, o, 0.0)


def _sc_dispatch_body(ids_hbm, x_hbm, xs_hbm, pos_hbm, meta_hbm,
                      ids_v, x_v, pos_v, hist_v, hist_all_v,
                      mt_tile_v, mt_exp_v, mt_lo_v, mt_hi_v, mt_init_v,
                      hist_sh, sem_x, sem_s):
    wid = lax.axis_index("s")
    base = wid * CHUNK
    lane = lax.iota(jnp.int32, LANES)
    zeros = jnp.zeros((LANES,), jnp.int32)

    cp_x = pltpu.async_copy(x_hbm.at[pl.ds(base, CHUNK), :], x_v, sem_x)
    pltpu.sync_copy(ids_hbm.at[pl.ds(base, CHUNK)], ids_v)

    hist = zeros
    for k in range(CHUNK // LANES):
        idv = ids_v[pl.ds(k * LANES, LANES)]
        for e in range(NG):
            cnt = jnp.sum(jnp.where(idv == e, 1, 0))
            hist = hist + jnp.where(lane == e, cnt, 0)
    hist_v[...] = hist

    pltpu.sync_copy(hist_v, hist_sh.at[pl.ds(wid * LANES, LANES)])
    plsc.subcore_barrier()
    pltpu.sync_copy(hist_sh, hist_all_v)

    tot = zeros
    pre = zeros
    for w2 in range(NW):
        row = hist_all_v[pl.ds(w2 * LANES, LANES)]
        tot = tot + row
        sel = jnp.full((LANES,), w2, jnp.int32) < wid
        pre = pre + jnp.where(sel, row, zeros)
    c_excl = plsc.cumsum(tot) - tot
    carry = c_excl + pre

    @pl.when(wid == 0)
    def _():
        mt_tile_v[...] = jnp.full((LANES,), T - 1, jnp.int32)
        mt_exp_v[...] = jnp.full((LANES,), NG - 1, jnp.int32)
        mt_lo_v[...] = zeros
        mt_hi_v[...] = zeros
        mt_init_v[...] = zeros
        cs = [jnp.sum(jnp.where(lane == e, c_excl, zeros)) for e in range(NG)]
        cs.append(jnp.asarray(M, jnp.int32))
        carry_s = jnp.asarray(0, jnp.int32)
        for v in range((T * NG + LANES - 1) // LANES):
            p = lane + v * LANES
            tt_v = p // NG
            ee_v = p % NG
            ce_v = zeros
            ce1_v = zeros
            for e in range(NG):
                ce_v = ce_v + jnp.where(ee_v == e, cs[e], 0)
                ce1_v = ce1_v + jnp.where(ee_v == e, cs[e + 1], 0)
            touched = (ce_v < (tt_v + 1) * TM) & (ce1_v > tt_v * TM)
            ti = jnp.where(touched, 1, 0)
            s_p = plsc.cumsum(ti) - ti + carry_s
            lo_v = jnp.maximum(ce_v - tt_v * TM, 0)
            hi_v = jnp.minimum(ce1_v - tt_v * TM, TM)
            plsc.store_scatter(mt_tile_v, [s_p], tt_v, mask=touched)
            plsc.store_scatter(mt_exp_v, [s_p], ee_v, mask=touched)
            plsc.store_scatter(mt_lo_v, [s_p], lo_v, mask=touched)
            plsc.store_scatter(mt_hi_v, [s_p], hi_v, mask=touched)
            plsc.store_scatter(mt_init_v, [s_p],
                               jnp.where(lo_v == 0, 1, 0), mask=touched)
            carry_s = carry_s + jnp.sum(ti)
        pltpu.sync_copy(mt_tile_v, meta_hbm.at[pl.ds(0 * LANES, LANES)])
        pltpu.sync_copy(mt_exp_v, meta_hbm.at[pl.ds(1 * LANES, LANES)])
        pltpu.sync_copy(mt_lo_v, meta_hbm.at[pl.ds(2 * LANES, LANES)])
        pltpu.sync_copy(mt_hi_v, meta_hbm.at[pl.ds(3 * LANES, LANES)])
        pltpu.sync_copy(mt_init_v, meta_hbm.at[pl.ds(4 * LANES, LANES)])

    for k in range(CHUNK // LANES):
        idv = ids_v[pl.ds(k * LANES, LANES)]
        posk = zeros
        vcnt = zeros
        for e in range(NG):
            mi = jnp.where(idv == e, 1, 0)
            pref = plsc.cumsum(mi) - mi
            ce = jnp.sum(jnp.where(lane == e, carry, zeros))
            posk = jnp.where(idv == e, ce + pref, posk)
            vcnt = vcnt + jnp.where(lane == e, jnp.sum(mi), 0)
        pos_v[pl.ds(k * LANES, LANES)] = posk
        carry = carry + vcnt

    cp_x.wait()
    pltpu.async_copy(x_v, xs_hbm.at[pos_v], sem_s).wait()
    pltpu.sync_copy(pos_v, pos_hbm.at[pl.ds(base, CHUNK)])


_sc_dispatch = functools.partial(
    pl.kernel,
    out_type=[
        jax.ShapeDtypeStruct((M, D), jnp.float32),
        jax.ShapeDtypeStruct((M,), jnp.int32),
        jax.ShapeDtypeStruct((5 * LANES,), jnp.int32),
    ],
    mesh=plsc.VectorSubcoreMesh(
        core_axis_name="c", subcore_axis_name="s", num_cores=1),
    compiler_params=pltpu.CompilerParams(needs_layout_passes=False),
    scratch_types=[
        pltpu.VMEM((CHUNK,), jnp.int32),
        pltpu.VMEM((CHUNK, D), jnp.float32),
        pltpu.VMEM((CHUNK,), jnp.int32),
        pltpu.VMEM((LANES,), jnp.int32),
        pltpu.VMEM((NW * LANES,), jnp.int32),
        pltpu.VMEM((LANES,), jnp.int32),
        pltpu.VMEM((LANES,), jnp.int32),
        pltpu.VMEM((LANES,), jnp.int32),
        pltpu.VMEM((LANES,), jnp.int32),
        pltpu.VMEM((LANES,), jnp.int32),
        pltpu.VMEM_SHARED((NW * LANES,), jnp.int32),
        pltpu.SemaphoreType.DMA,
        pltpu.SemaphoreType.DMA,
    ],
)(_sc_dispatch_body)


@jax.jit
def _run(x_BSD, Wg_DN, Wl1_ND2H, Wl2_NHD):
    ids3 = pl.pallas_call(
        _router_kernel,
        grid=(1,),
        in_specs=[
            pl.BlockSpec((1, M, D), lambda t: (0, 0, 0)),
            pl.BlockSpec((D, N), lambda t: (0, 0)),
        ],
        out_specs=pl.BlockSpec((1, 1, M), lambda t: (0, 0, 0)),
        out_shape=jax.ShapeDtypeStruct((1, 1, M), jnp.int32),
    )(x_BSD, Wg_DN)
    ids = ids3.reshape(M)

    x2 = x_BSD.reshape(M, D)
    xs, pos, meta = _sc_dispatch(ids, x2)

    y_ss = pl.pallas_call(
        _shared_kernel,
        grid=(T,),
        in_specs=[
            pl.BlockSpec((TM, D), lambda t: (t, 0)),
            pl.BlockSpec((1, D, 2 * H), lambda t: (NG, 0, 0)),
            pl.BlockSpec((1, H, D), lambda t: (NG, 0, 0)),
        ],
        out_specs=pl.BlockSpec((TM, D), lambda t: (t, 0)),
        out_shape=jax.ShapeDtypeStruct((M, D), jnp.float32),
    )(xs, Wl1_ND2H, Wl2_NHD)

    y_ts = pl.pallas_call(
        _gmm_kernel,
        grid_spec=pltpu.PrefetchScalarGridSpec(
            num_scalar_prefetch=1,
            grid=(LANES,),
            in_specs=[
                pl.BlockSpec((TM, D), lambda w, m: (m[w], 0)),
                pl.BlockSpec((TM, D), lambda w, m: (m[w], 0)),
                pl.BlockSpec((1, D, 2 * H), lambda w, m: (m[LANES + w], 0, 0)),
                pl.BlockSpec((1, H, D), lambda w, m: (m[LANES + w], 0, 0)),
            ],
            out_specs=pl.BlockSpec((TM, D), lambda w, m: (m[w], 0)),
        ),
        out_shape=jax.ShapeDtypeStruct((M, D), jnp.float32),
    )(meta, xs, y_ss, Wl1_ND2H, Wl2_NHD)

    y = y_ts.at[pos].get(mode="promise_in_bounds")
    return y.reshape(x_BSD.shape)


def kernel(x_BSD, Wg_DN, Wl1_ND2H, Wl2_NHD):
    return _run(x_BSD, Wg_DN, Wl1_ND2H, Wl2_NHD)
